# baseline (speedup 1.0000x reference)
import functools

import jax
import jax.numpy as jnp
from jax.experimental import pallas as pl
from jax.experimental.pallas import tpu as pltpu

_NUM_Q = 4
_N_E = 1024
_E_DIM = 256
_BETA = 0.25
_BLOCK = 512


def _rvq_block(x_ref, cb_ref, xq_ref, idx_ref, dist_ref, codes_ref, loss_ref,
               hi_s, mid_s, lo_s):
    @pl.when(pl.program_id(0) == 0)
    def _split():
        for i in range(_NUM_Q):
            cb = cb_ref[i]
            hi = cb.astype(jnp.bfloat16)
            rem = cb - hi.astype(jnp.float32)
            mid = rem.astype(jnp.bfloat16)
            lo = (rem - mid.astype(jnp.float32)).astype(jnp.bfloat16)
            hi_s[i] = hi
            mid_s[i] = mid
            lo_s[i] = lo

    r = x_ref[...]
    acc = jnp.zeros_like(r)
    loss = jnp.float32(0.0)
    idx_cols = []
    for i in range(_NUM_Q):
        cb = cb_ref[i]
        cb2 = jnp.sum(cb * cb, axis=1)
        r2 = jnp.sum(r * r, axis=1, keepdims=True)
        prod = jax.lax.dot_general(
            r, cb, (((1,), (1,)), ((), ())),
            preferred_element_type=jnp.float32,
        )
        d = r2 + cb2[None, :] - 2.0 * prod
        dist_ref[:, i, :] = d
        idx = jnp.argmin(d, axis=1).astype(jnp.int32)
        lane = jax.lax.broadcasted_iota(jnp.int32, d.shape, 1)
        idx_cols.append(idx)
        onehot = (lane == idx[:, None]).astype(jnp.bfloat16)
        dn = (((1,), (0,)), ((), ()))
        code = (
            jax.lax.dot_general(onehot, hi_s[i], dn,
                                preferred_element_type=jnp.float32)
            + jax.lax.dot_general(onehot, mid_s[i], dn,
                                  preferred_element_type=jnp.float32)
            + jax.lax.dot_general(onehot, lo_s[i], dn,
                                  preferred_element_type=jnp.float32)
        )
        codes_ref[:, i, :] = code
        diff = code - r
        loss = loss + jnp.sum(diff * diff)
        r = r - code
        acc = acc + code
    xq_ref[...] = acc
    idx_ref[...] = jnp.stack(idx_cols, axis=-1)
    loss_ref[0, 0, 0] = loss


def kernel(x, codebooks):
    n_tok, e_dim = x.shape
    num_q, n_e, _ = codebooks.shape
    num_blocks = n_tok // _BLOCK

    out_shapes = (
        jax.ShapeDtypeStruct((n_tok, e_dim), jnp.float32),
        jax.ShapeDtypeStruct((n_tok, num_q), jnp.int32),
        jax.ShapeDtypeStruct((n_tok, num_q, n_e), jnp.float32),
        jax.ShapeDtypeStruct((n_tok, num_q, e_dim), jnp.float32),
        jax.ShapeDtypeStruct((num_blocks, 1, 1), jnp.float32),
    )
    x_q, indices, distances, codes, loss_parts = pl.pallas_call(
        _rvq_block,
        grid=(num_blocks,),
        in_specs=[
            pl.BlockSpec((_BLOCK, e_dim), lambda i: (i, 0)),
            pl.BlockSpec((num_q, n_e, e_dim), lambda i: (0, 0, 0)),
        ],
        out_specs=[
            pl.BlockSpec((_BLOCK, e_dim), lambda i: (i, 0)),
            pl.BlockSpec((_BLOCK, num_q), lambda i: (i, 0)),
            pl.BlockSpec((_BLOCK, num_q, n_e), lambda i: (i, 0, 0)),
            pl.BlockSpec((_BLOCK, num_q, e_dim), lambda i: (i, 0, 0)),
            pl.BlockSpec((1, 1, 1), lambda i: (i, 0, 0), memory_space=pltpu.SMEM),
        ],
        out_shape=out_shapes,
        compiler_params=pltpu.CompilerParams(
            dimension_semantics=("arbitrary",),
        ),
        scratch_shapes=[
            pltpu.VMEM((num_q, n_e, e_dim), jnp.bfloat16),
            pltpu.VMEM((num_q, n_e, e_dim), jnp.bfloat16),
            pltpu.VMEM((num_q, n_e, e_dim), jnp.bfloat16),
        ],
    )(x, codebooks)
    scale = (1.0 + _BETA) / (num_q * n_tok * e_dim)
    mean_losses = jnp.sum(loss_parts) * scale
    return (x_q, mean_losses, indices, distances, codes)


# speedup vs baseline: 1.2717x; 1.2717x over previous
import functools

import jax
import jax.numpy as jnp
from jax.experimental import pallas as pl
from jax.experimental.pallas import tpu as pltpu

_NUM_Q = 4
_N_E = 1024
_E_DIM = 256
_BETA = 0.25
_BLOCK = 512


_HALF = _BLOCK // 2


def _rvq_block(x_ref, cb_ref, xq_ref, idx_ref, dist_ref, codes_ref, loss_ref,
               hi_s, mid_s, lo_s, cb2_s):
    @pl.when(pl.program_id(0) == 0)
    def _split():
        for i in range(_NUM_Q):
            cb = cb_ref[i]
            hi = cb.astype(jnp.bfloat16)
            rem = cb - hi.astype(jnp.float32)
            mid = rem.astype(jnp.bfloat16)
            lo = (rem - mid.astype(jnp.float32)).astype(jnp.bfloat16)
            hi_s[i] = hi
            mid_s[i] = mid
            lo_s[i] = lo
            del cb

    rs = [x_ref[pl.ds(h * _HALF, _HALF), :] for h in range(2)]
    accs = [jnp.zeros_like(rs[0]) for _ in range(2)]
    losses = [jnp.float32(0.0) for _ in range(2)]
    idx_cols = [[], []]
    dn = (((1,), (0,)), ((), ()))
    for i in range(_NUM_Q):
        cb = cb_ref[i]
        cb2 = jnp.sum(cb * cb, axis=1)
        for h in range(2):
            r = rs[h]
            r2 = jnp.sum(r * r, axis=1, keepdims=True)
            prod = jax.lax.dot_general(
                r, cb, (((1,), (1,)), ((), ())),
                preferred_element_type=jnp.float32,
            )
            d = r2 + cb2 - 2.0 * prod
            dist_ref[pl.ds(h * _HALF, _HALF), i, :] = d
            dmin = jnp.min(d, axis=1, keepdims=True)
            lane = jax.lax.broadcasted_iota(jnp.int32, d.shape, 1)
            idx = jnp.min(jnp.where(d == dmin, lane, _N_E), axis=1)
            idx_cols[h].append(idx)
            onehot = (lane == idx[:, None]).astype(jnp.bfloat16)
            code = (
                jax.lax.dot_general(onehot, hi_s[i], dn,
                                    preferred_element_type=jnp.float32)
                + jax.lax.dot_general(onehot, mid_s[i], dn,
                                      preferred_element_type=jnp.float32)
                + jax.lax.dot_general(onehot, lo_s[i], dn,
                                      preferred_element_type=jnp.float32)
            )
            codes_ref[pl.ds(h * _HALF, _HALF), i, :] = code
            diff = code - r
            losses[h] = losses[h] + jnp.sum(diff * diff)
            rs[h] = r - code
            accs[h] = accs[h] + code
    for h in range(2):
        xq_ref[pl.ds(h * _HALF, _HALF), :] = accs[h]
        idx_ref[pl.ds(h * _HALF, _HALF), :] = jnp.stack(idx_cols[h], axis=-1)
    loss_ref[0, 0, 0] = losses[0] + losses[1]


def kernel(x, codebooks):
    n_tok, e_dim = x.shape
    num_q, n_e, _ = codebooks.shape
    num_blocks = n_tok // _BLOCK

    out_shapes = (
        jax.ShapeDtypeStruct((n_tok, e_dim), jnp.float32),
        jax.ShapeDtypeStruct((n_tok, num_q), jnp.int32),
        jax.ShapeDtypeStruct((n_tok, num_q, n_e), jnp.float32),
        jax.ShapeDtypeStruct((n_tok, num_q, e_dim), jnp.float32),
        jax.ShapeDtypeStruct((num_blocks, 1, 1), jnp.float32),
    )
    x_q, indices, distances, codes, loss_parts = pl.pallas_call(
        _rvq_block,
        grid=(num_blocks,),
        in_specs=[
            pl.BlockSpec((_BLOCK, e_dim), lambda i: (i, 0)),
            pl.BlockSpec((num_q, n_e, e_dim), lambda i: (0, 0, 0)),
        ],
        out_specs=[
            pl.BlockSpec((_BLOCK, e_dim), lambda i: (i, 0)),
            pl.BlockSpec((_BLOCK, num_q), lambda i: (i, 0)),
            pl.BlockSpec((_BLOCK, num_q, n_e), lambda i: (i, 0, 0)),
            pl.BlockSpec((_BLOCK, num_q, e_dim), lambda i: (i, 0, 0)),
            pl.BlockSpec((1, 1, 1), lambda i: (i, 0, 0), memory_space=pltpu.SMEM),
        ],
        out_shape=out_shapes,
        compiler_params=pltpu.CompilerParams(
            dimension_semantics=("arbitrary",),
        ),
        scratch_shapes=[
            pltpu.VMEM((num_q, n_e, e_dim), jnp.bfloat16),
            pltpu.VMEM((num_q, n_e, e_dim), jnp.bfloat16),
            pltpu.VMEM((num_q, n_e, e_dim), jnp.bfloat16),
            pltpu.VMEM((num_q, n_e), jnp.float32),
        ],
    )(x, codebooks)
    scale = (1.0 + _BETA) / (num_q * n_tok * e_dim)
    mean_losses = jnp.sum(loss_parts) * scale
    return (x_q, mean_losses, indices, distances, codes)


# speedup vs baseline: 1.5228x; 1.1974x over previous
import functools

import jax
import jax.numpy as jnp
import numpy as np
from jax.experimental import pallas as pl
from jax.experimental.pallas import tpu as pltpu
from jax.sharding import Mesh, PartitionSpec as P

try:
    from jax import shard_map as _shard_map
except ImportError:
    from jax.experimental.shard_map import shard_map as _shard_map

_NUM_Q = 4
_N_E = 1024
_E_DIM = 256
_BETA = 0.25
_BLOCK = 512


_HALF = _BLOCK // 2


def _rvq_block(x_ref, cb_ref, xq_ref, idx_ref, dist_ref, codes_ref, loss_ref,
               hi_s, mid_s, lo_s, cb2_s):
    @pl.when(pl.program_id(0) == 0)
    def _split():
        for i in range(_NUM_Q):
            cb = cb_ref[i]
            hi = cb.astype(jnp.bfloat16)
            rem = cb - hi.astype(jnp.float32)
            mid = rem.astype(jnp.bfloat16)
            lo = (rem - mid.astype(jnp.float32)).astype(jnp.bfloat16)
            hi_s[i] = hi
            mid_s[i] = mid
            lo_s[i] = lo
            del cb

    rs = [x_ref[pl.ds(h * _HALF, _HALF), :] for h in range(2)]
    accs = [jnp.zeros_like(rs[0]) for _ in range(2)]
    losses = [jnp.float32(0.0) for _ in range(2)]
    idx_cols = [[], []]
    dn = (((1,), (0,)), ((), ()))
    for i in range(_NUM_Q):
        cb = cb_ref[i]
        cb2 = jnp.sum(cb * cb, axis=1)
        for h in range(2):
            r = rs[h]
            r2 = jnp.sum(r * r, axis=1, keepdims=True)
            prod = jax.lax.dot_general(
                r, cb, (((1,), (1,)), ((), ())),
                preferred_element_type=jnp.float32,
            )
            d = r2 + cb2 - 2.0 * prod
            dist_ref[pl.ds(h * _HALF, _HALF), i, :] = d
            dmin = jnp.min(d, axis=1, keepdims=True)
            lane = jax.lax.broadcasted_iota(jnp.int32, d.shape, 1)
            idx = jnp.min(jnp.where(d == dmin, lane, _N_E), axis=1)
            idx_cols[h].append(idx)
            onehot = (lane == idx[:, None]).astype(jnp.bfloat16)
            code = (
                jax.lax.dot_general(onehot, hi_s[i], dn,
                                    preferred_element_type=jnp.float32)
                + jax.lax.dot_general(onehot, mid_s[i], dn,
                                      preferred_element_type=jnp.float32)
                + jax.lax.dot_general(onehot, lo_s[i], dn,
                                      preferred_element_type=jnp.float32)
            )
            codes_ref[pl.ds(h * _HALF, _HALF), i, :] = code
            diff = code - r
            losses[h] = losses[h] + jnp.sum(diff * diff)
            rs[h] = r - code
            accs[h] = accs[h] + code
    for h in range(2):
        xq_ref[pl.ds(h * _HALF, _HALF), :] = accs[h]
        idx_ref[pl.ds(h * _HALF, _HALF), :] = jnp.stack(idx_cols[h], axis=-1)
    loss_ref[0, 0, 0] = losses[0] + losses[1]


def _rvq_call(x, codebooks):
    n_tok, e_dim = x.shape
    num_q, n_e, _ = codebooks.shape
    num_blocks = n_tok // _BLOCK

    out_shapes = (
        jax.ShapeDtypeStruct((n_tok, e_dim), jnp.float32),
        jax.ShapeDtypeStruct((n_tok, num_q), jnp.int32),
        jax.ShapeDtypeStruct((n_tok, num_q, n_e), jnp.float32),
        jax.ShapeDtypeStruct((n_tok, num_q, e_dim), jnp.float32),
        jax.ShapeDtypeStruct((num_blocks, 1, 1), jnp.float32),
    )
    x_q, indices, distances, codes, loss_parts = pl.pallas_call(
        _rvq_block,
        grid=(num_blocks,),
        in_specs=[
            pl.BlockSpec((_BLOCK, e_dim), lambda i: (i, 0)),
            pl.BlockSpec((num_q, n_e, e_dim), lambda i: (0, 0, 0)),
        ],
        out_specs=[
            pl.BlockSpec((_BLOCK, e_dim), lambda i: (i, 0)),
            pl.BlockSpec((_BLOCK, num_q), lambda i: (i, 0)),
            pl.BlockSpec((_BLOCK, num_q, n_e), lambda i: (i, 0, 0)),
            pl.BlockSpec((_BLOCK, num_q, e_dim), lambda i: (i, 0, 0)),
            pl.BlockSpec((1, 1, 1), lambda i: (i, 0, 0), memory_space=pltpu.SMEM),
        ],
        out_shape=out_shapes,
        compiler_params=pltpu.CompilerParams(
            dimension_semantics=("arbitrary",),
        ),
        scratch_shapes=[
            pltpu.VMEM((num_q, n_e, e_dim), jnp.bfloat16),
            pltpu.VMEM((num_q, n_e, e_dim), jnp.bfloat16),
            pltpu.VMEM((num_q, n_e, e_dim), jnp.bfloat16),
            pltpu.VMEM((num_q, n_e), jnp.float32),
        ],
    )(x, codebooks)
    return (x_q, indices, distances, codes, loss_parts)


def kernel(x, codebooks):
    n_tok, e_dim = x.shape
    num_q = codebooks.shape[0]
    devs = jax.devices()
    n_dev = len(devs)
    if n_dev > 1 and n_tok % (n_dev * _BLOCK) == 0:
        mesh = Mesh(np.array(devs), ("d",))
        f = _shard_map(
            _rvq_call,
            mesh=mesh,
            in_specs=(P("d", None), P(None, None, None)),
            out_specs=(P("d", None), P("d", None), P("d", None, None),
                       P("d", None, None), P("d", None, None)),
            check_vma=False,
        )
        x_q, indices, distances, codes, loss_parts = f(x, codebooks)
    else:
        x_q, indices, distances, codes, loss_parts = _rvq_call(x, codebooks)
    scale = (1.0 + _BETA) / (num_q * n_tok * e_dim)
    mean_losses = jnp.sum(loss_parts) * scale
    return (x_q, mean_losses, indices, distances, codes)
